# baseline (speedup 1.0000x reference)
import jax
import jax.numpy as jnp
from jax.experimental import pallas as pl
from jax.experimental.pallas import tpu as pltpu


def _stem_body(a_ref, w_hbm, s_hbm, b_hbm, o_ref,
               wv, sv, bv, slab, sem):
    @pl.when(pl.program_id(0) == 0)
    def _():
        pltpu.make_async_copy(w_hbm, wv, sem).start()
        pltpu.make_async_copy(s_hbm, sv, sem).start()
        pltpu.make_async_copy(b_hbm, bv, sem).start()
        pltpu.make_async_copy(w_hbm, wv, sem).wait()
        pltpu.make_async_copy(s_hbm, sv, sem).wait()
        pltpu.make_async_copy(b_hbm, bv, sem).wait()
        slab[...] = jnp.full_like(slab, -jnp.inf)

    for wh in range(2):
        a = a_ref[:, :, 112 * wh:112 * wh + 112]
        acc = jax.lax.dot_general(a, wv[...], (((0,), (0,)), ((), ())),
                                  preferred_element_type=jnp.float32)
        c1 = jnp.maximum(acc * sv[...] + bv[...], 0.0)
        slab[1:113, 2 + 112 * wh:114 + 112 * wh, :] = c1

    m = None
    for dh in range(3):
        for dw in range(3):
            t = slab[dh:dh + 112:2, 2 * dw:2 * dw + 221:4, :]
            m = t if m is None else jnp.maximum(m, t)
    o_ref[...] = m.astype(o_ref.dtype)


def _stem(x_nchw, conv1_w, bn1_scale, bn1_bias):
    xb = jnp.pad(x_nchw.astype(jnp.bfloat16),
                 ((0, 0), (0, 0), (3, 3), (3, 3)))
    a = jnp.concatenate(
        [xb[:, :, kh:kh + 223:2, kw:kw + 224]
         for kh in range(7) for kw in range(7)], axis=1)
    wf = jnp.transpose(conv1_w, (0, 1, 2, 3)).reshape(147, 64)

    return pl.pallas_call(
        _stem_body,
        out_shape=jax.ShapeDtypeStruct((32, 56, 56, 64), jnp.bfloat16),
        grid=(32,),
        in_specs=[
            pl.BlockSpec((None, 147, 112, 224), lambda n: (n, 0, 0, 0)),
            pl.BlockSpec(memory_space=pl.ANY),
            pl.BlockSpec(memory_space=pl.ANY),
            pl.BlockSpec(memory_space=pl.ANY),
        ],
        out_specs=pl.BlockSpec((None, 56, 56, 64), lambda n: (n, 0, 0, 0)),
        scratch_shapes=[
            pltpu.VMEM((147, 64), jnp.bfloat16),
            pltpu.VMEM((1, 64), jnp.float32),
            pltpu.VMEM((1, 64), jnp.float32),
            pltpu.VMEM((114, 228, 64), jnp.float32),
            pltpu.SemaphoreType.DMA,
        ],
        compiler_params=pltpu.CompilerParams(
            dimension_semantics=("arbitrary",)),
    )(a, wf, bn1_scale, bn1_bias)


def _make_stage_body(nw, stride, n_blocks, H, W, Cm, Cin):
    OH, OW = H // stride, W // stride

    def body(x_ref, *rest):
        whbm = rest[:nw]
        o_ref = rest[nw]
        wv = rest[nw + 1:nw + 1 + nw]
        slab = rest[nw + 1 + nw]
        extra = rest[nw + 2 + nw:-1]
        sem = rest[-1]

        @pl.when(pl.program_id(0) == 0)
        def _():
            for s, d in zip(whbm, wv):
                pltpu.make_async_copy(s, d, sem).start()
            for s, d in zip(whbm, wv):
                pltpu.make_async_copy(s, d, sem).wait()
            slab[...] = jnp.zeros_like(slab)
            if stride == 2:
                extra[0][...] = jnp.zeros_like(extra[0])

        v = x_ref[...]
        wi = 0

        def nxt():
            nonlocal wi
            r = wv[wi]
            wi += 1
            return r

        for b in range(n_blocks):
            w1, s1, b1 = nxt(), nxt(), nxt()
            w2, s2, b2 = nxt(), nxt(), nxt()
            w3, s3, b3 = nxt(), nxt(), nxt()
            has_ds = b == 0
            if has_ds:
                wd, sd, bd = nxt(), nxt(), nxt()

            a1 = jax.lax.dot_general(v, w1[...], (((2,), (0,)), ((), ())),
                                     preferred_element_type=jnp.float32)
            a1 = jnp.maximum(a1 * s1[...] + b1[...], 0.0)

            st = stride if has_ds else 1
            if st == 2:
                slab32, xf32 = extra
                cm = min(Cm, 128)
                ncm = Cm // cm
                for c in range(ncm):
                    slab32[c, 1:H + 1, 1:W + 1, :] = a1[:, :, c * cm:(c + 1) * cm]
                t = jnp.concatenate(
                    [slab32[c, kh:kh + 2 * OH:2, kw:kw + 2 * OW:2, :]
                     for kh in range(3) for kw in range(3)
                     for c in range(ncm)], axis=-1).astype(jnp.bfloat16)
            else:
                slab[1:OH + 1, 1:OW + 1, :Cm] = a1.astype(jnp.bfloat16)
                Cs = slab.shape[2]
                t = jnp.concatenate(
                    [slab[kh:kh + OH, kw:kw + OW, :]
                     for kh in range(3) for kw in range(3)], axis=-1)
            acc = jax.lax.dot_general(t, w2[...], (((2,), (0,)), ((), ())),
                                      preferred_element_type=jnp.float32)
            a2 = jnp.maximum(acc * s2[...] + b2[...], 0.0).astype(jnp.bfloat16)

            if has_ds:
                if st == 2:
                    cin = min(Cin, 128)
                    ncin = Cin // cin
                    vf = v.astype(jnp.float32)
                    for c in range(ncin):
                        xf32[c] = vf[:, :, c * cin:(c + 1) * cin]
                    xs = jnp.concatenate(
                        [xf32[c, 0:2 * OH:2, 0:2 * OW:2, :]
                         for c in range(ncin)],
                        axis=-1).astype(jnp.bfloat16)
                else:
                    xs = v
                res = jax.lax.dot_general(xs, wd[...],
                                          (((2,), (0,)), ((), ())),
                                          preferred_element_type=jnp.float32)
                res = res * sd[...] + bd[...]
            else:
                res = v.astype(jnp.float32)

            z = jax.lax.dot_general(a2, w3[...], (((2,), (0,)), ((), ())),
                                    preferred_element_type=jnp.float32)
            z = jnp.maximum(z * s3[...] + b3[...] + res, 0.0)
            v = z.astype(jnp.bfloat16)

        o_ref[...] = v

    return body


def _stage(x, blocks, stride):
    N, H, W, Cin = x.shape
    Cm = blocks[0][0].shape[1]
    C4 = 4 * Cm
    OH, OW = H // stride, W // stride

    Cs_w = max(Cm, 128)
    tblocks = []
    for bi, blk in enumerate(blocks):
        blk = list(blk)
        w2 = blk[3]
        if stride == 2 and bi == 0:
            blk[3] = w2.reshape(9 * Cm, Cm)
        else:
            w2p = jnp.pad(w2, ((0, 0), (0, 0), (0, Cs_w - Cm), (0, 0)))
            blk[3] = w2p.reshape(9 * Cs_w, Cm)
        tblocks.append(tuple(blk))
    flat = [a for blk in tblocks for a in blk]
    body = _make_stage_body(len(flat), stride, len(blocks), H, W, Cm, Cin)

    scratch = [pltpu.VMEM(a.shape, a.dtype) for a in flat]
    Cs = max(Cm, 128)
    scratch.append(pltpu.VMEM((OH + 2, OW + 2, Cs), jnp.bfloat16))
    if stride == 2:
        cm = min(Cm, 128)
        cin = min(Cin, 128)
        scratch.append(pltpu.VMEM((Cm // cm, H + 2, W + 2, cm), jnp.float32))
        scratch.append(pltpu.VMEM((Cin // cin, H, W, cin), jnp.float32))
    scratch.append(pltpu.SemaphoreType.DMA)

    return pl.pallas_call(
        body,
        out_shape=jax.ShapeDtypeStruct((N, OH, OW, C4), jnp.bfloat16),
        grid=(N,),
        in_specs=[pl.BlockSpec((None, H, W, Cin), lambda n: (n, 0, 0, 0))] +
                 [pl.BlockSpec(memory_space=pl.ANY)] * len(flat),
        out_specs=pl.BlockSpec((None, OH, OW, C4), lambda n: (n, 0, 0, 0)),
        scratch_shapes=scratch,
        compiler_params=pltpu.CompilerParams(
            dimension_semantics=("arbitrary",)),
    )(x, *flat)


def _head_body(x_ref, w_ref, b_ref, o_ref):
    feat = jnp.mean(x_ref[...].astype(jnp.float32), axis=1)
    z = jnp.dot(feat, w_ref[...],
                preferred_element_type=jnp.float32) + b_ref[...]
    z = z - jnp.max(z, axis=1, keepdims=True)
    e = jnp.exp(z)
    o_ref[...] = e / jnp.sum(e, axis=1, keepdims=True)


def _head(x, fc_w, fc_b):
    xr = x.reshape(32, 49, 2048)
    return pl.pallas_call(
        _head_body,
        out_shape=jax.ShapeDtypeStruct((32, 1000), jnp.float32),
        grid=(1,),
        in_specs=[pl.BlockSpec((32, 49, 2048), lambda i: (0, 0, 0)),
                  pl.BlockSpec((2048, 1000), lambda i: (0, 0)),
                  pl.BlockSpec((1, 1000), lambda i: (0, 0))],
        out_specs=pl.BlockSpec((32, 1000), lambda i: (0, 0)),
    )(xr, fc_w, fc_b)


def kernel(
    x_nchw, conv1_w, bn1_scale, bn1_bias,
    l0b0_w1, l0b0_bn1s, l0b0_bn1b, l0b0_w2, l0b0_bn2s, l0b0_bn2b,
    l0b0_w3, l0b0_bn3s, l0b0_bn3b, l0b0_wd, l0b0_bnds, l0b0_bndb,
    l0b1_w1, l0b1_bn1s, l0b1_bn1b, l0b1_w2, l0b1_bn2s, l0b1_bn2b,
    l0b1_w3, l0b1_bn3s, l0b1_bn3b,
    l0b2_w1, l0b2_bn1s, l0b2_bn1b, l0b2_w2, l0b2_bn2s, l0b2_bn2b,
    l0b2_w3, l0b2_bn3s, l0b2_bn3b,
    l1b0_w1, l1b0_bn1s, l1b0_bn1b, l1b0_w2, l1b0_bn2s, l1b0_bn2b,
    l1b0_w3, l1b0_bn3s, l1b0_bn3b, l1b0_wd, l1b0_bnds, l1b0_bndb,
    l1b1_w1, l1b1_bn1s, l1b1_bn1b, l1b1_w2, l1b1_bn2s, l1b1_bn2b,
    l1b1_w3, l1b1_bn3s, l1b1_bn3b,
    l1b2_w1, l1b2_bn1s, l1b2_bn1b, l1b2_w2, l1b2_bn2s, l1b2_bn2b,
    l1b2_w3, l1b2_bn3s, l1b2_bn3b,
    l1b3_w1, l1b3_bn1s, l1b3_bn1b, l1b3_w2, l1b3_bn2s, l1b3_bn2b,
    l1b3_w3, l1b3_bn3s, l1b3_bn3b,
    l2b0_w1, l2b0_bn1s, l2b0_bn1b, l2b0_w2, l2b0_bn2s, l2b0_bn2b,
    l2b0_w3, l2b0_bn3s, l2b0_bn3b, l2b0_wd, l2b0_bnds, l2b0_bndb,
    l2b1_w1, l2b1_bn1s, l2b1_bn1b, l2b1_w2, l2b1_bn2s, l2b1_bn2b,
    l2b1_w3, l2b1_bn3s, l2b1_bn3b,
    l2b2_w1, l2b2_bn1s, l2b2_bn1b, l2b2_w2, l2b2_bn2s, l2b2_bn2b,
    l2b2_w3, l2b2_bn3s, l2b2_bn3b,
    l2b3_w1, l2b3_bn1s, l2b3_bn1b, l2b3_w2, l2b3_bn2s, l2b3_bn2b,
    l2b3_w3, l2b3_bn3s, l2b3_bn3b,
    l2b4_w1, l2b4_bn1s, l2b4_bn1b, l2b4_w2, l2b4_bn2s, l2b4_bn2b,
    l2b4_w3, l2b4_bn3s, l2b4_bn3b,
    l2b5_w1, l2b5_bn1s, l2b5_bn1b, l2b5_w2, l2b5_bn2s, l2b5_bn2b,
    l2b5_w3, l2b5_bn3s, l2b5_bn3b,
    l3b0_w1, l3b0_bn1s, l3b0_bn1b, l3b0_w2, l3b0_bn2s, l3b0_bn2b,
    l3b0_w3, l3b0_bn3s, l3b0_bn3b, l3b0_wd, l3b0_bnds, l3b0_bndb,
    l3b1_w1, l3b1_bn1s, l3b1_bn1b, l3b1_w2, l3b1_bn2s, l3b1_bn2b,
    l3b1_w3, l3b1_bn3s, l3b1_bn3b,
    l3b2_w1, l3b2_bn1s, l3b2_bn1b, l3b2_w2, l3b2_bn2s, l3b2_bn2b,
    l3b2_w3, l3b2_bn3s, l3b2_bn3b,
    fc_w, fc_b,
):
    g = dict(locals())

    def blk(p, ds):
        ks = ["w1", "bn1s", "bn1b", "w2", "bn2s", "bn2b", "w3", "bn3s", "bn3b"]
        if ds:
            ks += ["wd", "bnds", "bndb"]
        out = []
        for k in ks:
            a = g[p + k]
            if k in ("w1", "w3", "wd"):
                a = a.reshape(a.shape[2], a.shape[3])
            out.append(a)
        return tuple(out)

    h = _stem(x_nchw, conv1_w, bn1_scale, bn1_bias)
    h = _stage(h, [blk("l0b0_", True), blk("l0b1_", False),
                   blk("l0b2_", False)], 1)
    h = _stage(h, [blk("l1b0_", True), blk("l1b1_", False),
                   blk("l1b2_", False), blk("l1b3_", False)], 2)
    h = _stage(h, [blk("l2b0_", True), blk("l2b1_", False),
                   blk("l2b2_", False), blk("l2b3_", False),
                   blk("l2b4_", False), blk("l2b5_", False)], 2)
    h = _stage(h, [blk("l3b0_", True), blk("l3b1_", False),
                   blk("l3b2_", False)], 2)
    return _head(h, fc_w, fc_b)


# speedup vs baseline: 1.5559x; 1.5559x over previous
import jax
import jax.numpy as jnp
from jax.experimental import pallas as pl
from jax.experimental.pallas import tpu as pltpu


def _stem_body(ae_ref, ao_ref, w_hbm, s_hbm, b_hbm, o_ref,
               wv, sv, bv, slab, sem):
    @pl.when(pl.program_id(0) == 0)
    def _():
        pltpu.make_async_copy(w_hbm, wv, sem).start()
        pltpu.make_async_copy(s_hbm, sv, sem).start()
        pltpu.make_async_copy(b_hbm, bv, sem).start()
        pltpu.make_async_copy(w_hbm, wv, sem).wait()
        pltpu.make_async_copy(s_hbm, sv, sem).wait()
        pltpu.make_async_copy(b_hbm, bv, sem).wait()
        slab[...] = jnp.full_like(slab, -jnp.inf)

    acc = None
    for kh in range(7):
        if kh % 2 == 0:
            a = ae_ref[kh // 2:kh // 2 + 112]
        else:
            a = ao_ref[(kh - 1) // 2:(kh - 1) // 2 + 112]
        d = jax.lax.dot_general(a, wv[kh], (((2,), (0,)), ((), ())),
                                preferred_element_type=jnp.float32)
        acc = d if acc is None else acc + d

    c1 = jnp.maximum(acc * sv[...] + bv[...], 0.0)
    slab[1:113, 1:113, :] = c1

    m = None
    for kh in range(3):
        for kw in range(3):
            t = slab[kh:kh + 112:2, kw:kw + 112:2, :]
            m = t if m is None else jnp.maximum(m, t)
    o_ref[...] = m.astype(o_ref.dtype)


def _stem(x_nchw, conv1_w, bn1_scale, bn1_bias):
    xt = jnp.transpose(x_nchw, (0, 2, 3, 1)).astype(jnp.bfloat16)
    xp = jnp.pad(xt, ((0, 0), (3, 3), (3, 2), (0, 0)))
    xpe = xp[:, 0::2]
    xpo = xp[:, 1::2]
    ae = jnp.concatenate(
        [xpe[:, :, kw:kw + 223:2, :] for kw in range(7)], axis=-1)
    ao = jnp.concatenate(
        [xpo[:, :, kw:kw + 223:2, :] for kw in range(7)], axis=-1)
    wf = conv1_w.reshape(7, 21, 64)

    return pl.pallas_call(
        _stem_body,
        out_shape=jax.ShapeDtypeStruct((32, 56, 56, 64), jnp.bfloat16),
        grid=(32,),
        in_specs=[
            pl.BlockSpec((None, 115, 112, 21), lambda n: (n, 0, 0, 0)),
            pl.BlockSpec((None, 115, 112, 21), lambda n: (n, 0, 0, 0)),
            pl.BlockSpec(memory_space=pl.ANY),
            pl.BlockSpec(memory_space=pl.ANY),
            pl.BlockSpec(memory_space=pl.ANY),
        ],
        out_specs=pl.BlockSpec((None, 56, 56, 64), lambda n: (n, 0, 0, 0)),
        scratch_shapes=[
            pltpu.VMEM((7, 21, 64), jnp.bfloat16),
            pltpu.VMEM((1, 64), jnp.float32),
            pltpu.VMEM((1, 64), jnp.float32),
            pltpu.VMEM((114, 114, 64), jnp.float32),
            pltpu.SemaphoreType.DMA,
        ],
        compiler_params=pltpu.CompilerParams(
            dimension_semantics=("arbitrary",)),
    )(ae, ao, wf, bn1_scale, bn1_bias)


def _make_stage_body(nw, stride, n_blocks, H, W, Cm, Cin):
    OH, OW = H // stride, W // stride

    def body(x_ref, *rest):
        whbm = rest[:nw]
        o_ref = rest[nw]
        wv = rest[nw + 1:nw + 1 + nw]
        slab = rest[nw + 1 + nw]
        extra = rest[nw + 2 + nw:-1]
        sem = rest[-1]

        @pl.when(pl.program_id(0) == 0)
        def _():
            for s, d in zip(whbm, wv):
                pltpu.make_async_copy(s, d, sem).start()
            for s, d in zip(whbm, wv):
                pltpu.make_async_copy(s, d, sem).wait()
            slab[...] = jnp.zeros_like(slab)
            if stride == 2:
                extra[0][...] = jnp.zeros_like(extra[0])

        v = x_ref[...]
        wi = 0

        def nxt():
            nonlocal wi
            r = wv[wi]
            wi += 1
            return r

        for b in range(n_blocks):
            w1, s1, b1 = nxt(), nxt(), nxt()
            w2, s2, b2 = nxt(), nxt(), nxt()
            w3, s3, b3 = nxt(), nxt(), nxt()
            has_ds = b == 0
            if has_ds:
                wd, sd, bd = nxt(), nxt(), nxt()

            a1 = jax.lax.dot_general(v, w1[...], (((2,), (0,)), ((), ())),
                                     preferred_element_type=jnp.float32)
            a1 = jnp.maximum(a1 * s1[...] + b1[...], 0.0)

            st = stride if has_ds else 1
            if st == 2:
                slab32, xf32 = extra
                cm = min(Cm, 128)
                ncm = Cm // cm
                for c in range(ncm):
                    slab32[c, 1:H + 1, 1:W + 1, :] = a1[:, :, c * cm:(c + 1) * cm]
                t = jnp.concatenate(
                    [slab32[c, kh:kh + 2 * OH:2, kw:kw + 2 * OW:2, :]
                     for kh in range(3) for kw in range(3)
                     for c in range(ncm)], axis=-1).astype(jnp.bfloat16)
            else:
                slab[1:OH + 1, 1:OW + 1, :Cm] = a1.astype(jnp.bfloat16)
                Cs = slab.shape[2]
                t = jnp.concatenate(
                    [slab[kh:kh + OH, kw:kw + OW, :]
                     for kh in range(3) for kw in range(3)], axis=-1)
            acc = jax.lax.dot_general(t, w2[...], (((2,), (0,)), ((), ())),
                                      preferred_element_type=jnp.float32)
            a2 = jnp.maximum(acc * s2[...] + b2[...], 0.0).astype(jnp.bfloat16)

            if has_ds:
                if st == 2:
                    cin = min(Cin, 128)
                    ncin = Cin // cin
                    vf = v.astype(jnp.float32)
                    for c in range(ncin):
                        xf32[c] = vf[:, :, c * cin:(c + 1) * cin]
                    xs = jnp.concatenate(
                        [xf32[c, 0:2 * OH:2, 0:2 * OW:2, :]
                         for c in range(ncin)],
                        axis=-1).astype(jnp.bfloat16)
                else:
                    xs = v
                res = jax.lax.dot_general(xs, wd[...],
                                          (((2,), (0,)), ((), ())),
                                          preferred_element_type=jnp.float32)
                res = res * sd[...] + bd[...]
            else:
                res = v.astype(jnp.float32)

            z = jax.lax.dot_general(a2, w3[...], (((2,), (0,)), ((), ())),
                                    preferred_element_type=jnp.float32)
            z = jnp.maximum(z * s3[...] + b3[...] + res, 0.0)
            v = z.astype(jnp.bfloat16)

        o_ref[...] = v

    return body


def _stage(x, blocks, stride):
    N, H, W, Cin = x.shape
    Cm = blocks[0][0].shape[1]
    C4 = 4 * Cm
    OH, OW = H // stride, W // stride

    Cs_w = max(Cm, 128)
    tblocks = []
    for bi, blk in enumerate(blocks):
        blk = list(blk)
        w2 = blk[3]
        if stride == 2 and bi == 0:
            blk[3] = w2.reshape(9 * Cm, Cm)
        else:
            w2p = jnp.pad(w2, ((0, 0), (0, 0), (0, Cs_w - Cm), (0, 0)))
            blk[3] = w2p.reshape(9 * Cs_w, Cm)
        tblocks.append(tuple(blk))
    flat = [a for blk in tblocks for a in blk]
    body = _make_stage_body(len(flat), stride, len(blocks), H, W, Cm, Cin)

    scratch = [pltpu.VMEM(a.shape, a.dtype) for a in flat]
    Cs = max(Cm, 128)
    scratch.append(pltpu.VMEM((OH + 2, OW + 2, Cs), jnp.bfloat16))
    if stride == 2:
        cm = min(Cm, 128)
        cin = min(Cin, 128)
        scratch.append(pltpu.VMEM((Cm // cm, H + 2, W + 2, cm), jnp.float32))
        scratch.append(pltpu.VMEM((Cin // cin, H, W, cin), jnp.float32))
    scratch.append(pltpu.SemaphoreType.DMA)

    return pl.pallas_call(
        body,
        out_shape=jax.ShapeDtypeStruct((N, OH, OW, C4), jnp.bfloat16),
        grid=(N,),
        in_specs=[pl.BlockSpec((None, H, W, Cin), lambda n: (n, 0, 0, 0))] +
                 [pl.BlockSpec(memory_space=pl.ANY)] * len(flat),
        out_specs=pl.BlockSpec((None, OH, OW, C4), lambda n: (n, 0, 0, 0)),
        scratch_shapes=scratch,
        compiler_params=pltpu.CompilerParams(
            dimension_semantics=("arbitrary",)),
    )(x, *flat)


def _head_body(x_ref, w_ref, b_ref, o_ref):
    feat = jnp.mean(x_ref[...].astype(jnp.float32), axis=1)
    z = jnp.dot(feat, w_ref[...],
                preferred_element_type=jnp.float32) + b_ref[...]
    z = z - jnp.max(z, axis=1, keepdims=True)
    e = jnp.exp(z)
    o_ref[...] = e / jnp.sum(e, axis=1, keepdims=True)


def _head(x, fc_w, fc_b):
    xr = x.reshape(32, 49, 2048)
    return pl.pallas_call(
        _head_body,
        out_shape=jax.ShapeDtypeStruct((32, 1000), jnp.float32),
        grid=(1,),
        in_specs=[pl.BlockSpec((32, 49, 2048), lambda i: (0, 0, 0)),
                  pl.BlockSpec((2048, 1000), lambda i: (0, 0)),
                  pl.BlockSpec((1, 1000), lambda i: (0, 0))],
        out_specs=pl.BlockSpec((32, 1000), lambda i: (0, 0)),
    )(xr, fc_w, fc_b)


def kernel(
    x_nchw, conv1_w, bn1_scale, bn1_bias,
    l0b0_w1, l0b0_bn1s, l0b0_bn1b, l0b0_w2, l0b0_bn2s, l0b0_bn2b,
    l0b0_w3, l0b0_bn3s, l0b0_bn3b, l0b0_wd, l0b0_bnds, l0b0_bndb,
    l0b1_w1, l0b1_bn1s, l0b1_bn1b, l0b1_w2, l0b1_bn2s, l0b1_bn2b,
    l0b1_w3, l0b1_bn3s, l0b1_bn3b,
    l0b2_w1, l0b2_bn1s, l0b2_bn1b, l0b2_w2, l0b2_bn2s, l0b2_bn2b,
    l0b2_w3, l0b2_bn3s, l0b2_bn3b,
    l1b0_w1, l1b0_bn1s, l1b0_bn1b, l1b0_w2, l1b0_bn2s, l1b0_bn2b,
    l1b0_w3, l1b0_bn3s, l1b0_bn3b, l1b0_wd, l1b0_bnds, l1b0_bndb,
    l1b1_w1, l1b1_bn1s, l1b1_bn1b, l1b1_w2, l1b1_bn2s, l1b1_bn2b,
    l1b1_w3, l1b1_bn3s, l1b1_bn3b,
    l1b2_w1, l1b2_bn1s, l1b2_bn1b, l1b2_w2, l1b2_bn2s, l1b2_bn2b,
    l1b2_w3, l1b2_bn3s, l1b2_bn3b,
    l1b3_w1, l1b3_bn1s, l1b3_bn1b, l1b3_w2, l1b3_bn2s, l1b3_bn2b,
    l1b3_w3, l1b3_bn3s, l1b3_bn3b,
    l2b0_w1, l2b0_bn1s, l2b0_bn1b, l2b0_w2, l2b0_bn2s, l2b0_bn2b,
    l2b0_w3, l2b0_bn3s, l2b0_bn3b, l2b0_wd, l2b0_bnds, l2b0_bndb,
    l2b1_w1, l2b1_bn1s, l2b1_bn1b, l2b1_w2, l2b1_bn2s, l2b1_bn2b,
    l2b1_w3, l2b1_bn3s, l2b1_bn3b,
    l2b2_w1, l2b2_bn1s, l2b2_bn1b, l2b2_w2, l2b2_bn2s, l2b2_bn2b,
    l2b2_w3, l2b2_bn3s, l2b2_bn3b,
    l2b3_w1, l2b3_bn1s, l2b3_bn1b, l2b3_w2, l2b3_bn2s, l2b3_bn2b,
    l2b3_w3, l2b3_bn3s, l2b3_bn3b,
    l2b4_w1, l2b4_bn1s, l2b4_bn1b, l2b4_w2, l2b4_bn2s, l2b4_bn2b,
    l2b4_w3, l2b4_bn3s, l2b4_bn3b,
    l2b5_w1, l2b5_bn1s, l2b5_bn1b, l2b5_w2, l2b5_bn2s, l2b5_bn2b,
    l2b5_w3, l2b5_bn3s, l2b5_bn3b,
    l3b0_w1, l3b0_bn1s, l3b0_bn1b, l3b0_w2, l3b0_bn2s, l3b0_bn2b,
    l3b0_w3, l3b0_bn3s, l3b0_bn3b, l3b0_wd, l3b0_bnds, l3b0_bndb,
    l3b1_w1, l3b1_bn1s, l3b1_bn1b, l3b1_w2, l3b1_bn2s, l3b1_bn2b,
    l3b1_w3, l3b1_bn3s, l3b1_bn3b,
    l3b2_w1, l3b2_bn1s, l3b2_bn1b, l3b2_w2, l3b2_bn2s, l3b2_bn2b,
    l3b2_w3, l3b2_bn3s, l3b2_bn3b,
    fc_w, fc_b,
):
    g = dict(locals())

    def blk(p, ds):
        ks = ["w1", "bn1s", "bn1b", "w2", "bn2s", "bn2b", "w3", "bn3s", "bn3b"]
        if ds:
            ks += ["wd", "bnds", "bndb"]
        out = []
        for k in ks:
            a = g[p + k]
            if k in ("w1", "w3", "wd"):
                a = a.reshape(a.shape[2], a.shape[3])
            out.append(a)
        return tuple(out)

    h = _stem(x_nchw, conv1_w, bn1_scale, bn1_bias)
    h = _stage(h, [blk("l0b0_", True), blk("l0b1_", False),
                   blk("l0b2_", False)], 1)
    h = _stage(h, [blk("l1b0_", True), blk("l1b1_", False),
                   blk("l1b2_", False), blk("l1b3_", False)], 2)
    h = _stage(h, [blk("l2b0_", True), blk("l2b1_", False),
                   blk("l2b2_", False), blk("l2b3_", False),
                   blk("l2b4_", False), blk("l2b5_", False)], 2)
    h = _stage(h, [blk("l3b0_", True), blk("l3b1_", False),
                   blk("l3b2_", False)], 2)
    return _head(h, fc_w, fc_b)
